# baseline (speedup 1.0000x reference)
import functools

import jax
import jax.numpy as jnp
from jax import lax
from jax.experimental import pallas as pl
from jax.experimental.pallas import tpu as pltpu
from jax.experimental.pallas import tpu_sc as plsc

D_MODEL = 128
EPS = 1e-5
_CHUNK = 128


def _make_sc_gather(n_rows: int, d: int):
    info = plsc.get_sparse_core_info()
    nc, ns = info.num_cores, info.num_subcores
    nw = nc * ns
    assert n_rows % nw == 0
    b_per_w = n_rows // nw
    offs = list(range(0, b_per_w, _CHUNK))
    szs = [min(_CHUNK, b_per_w - o) for o in offs]
    n = len(offs)
    mesh = plsc.VectorSubcoreMesh(core_axis_name="c", subcore_axis_name="s")

    @functools.partial(
        pl.kernel,
        mesh=mesh,
        out_type=jax.ShapeDtypeStruct((n_rows, d), jnp.float32),
        scratch_types=[
            pltpu.VMEM((b_per_w,), jnp.int32),
            pltpu.VMEM((_CHUNK, d), jnp.float32),
            pltpu.VMEM((_CHUNK, d), jnp.float32),
            pltpu.SemaphoreType.DMA,
            pltpu.SemaphoreType.DMA,
            pltpu.SemaphoreType.DMA,
            pltpu.SemaphoreType.DMA,
        ],
    )
    def gather_kernel(table_hbm, idx_hbm, out_hbm, idx_v, buf_a, buf_b,
                      gs_a, gs_b, os_a, os_b):
        wid = lax.axis_index("s") * nc + lax.axis_index("c")
        base = wid * b_per_w
        pltpu.sync_copy(idx_hbm.at[pl.ds(base, b_per_w)], idx_v)
        bufs = (buf_a, buf_b)
        gsems = (gs_a, gs_b)
        osems = (os_a, os_b)

        def start_gather(k):
            i = k % 2
            h = pltpu.make_async_copy(
                table_hbm.at[idx_v.at[pl.ds(offs[k], szs[k])]],
                bufs[i].at[pl.ds(0, szs[k])],
                gsems[i])
            h.start()
            return h

        def start_out(k):
            i = k % 2
            h = pltpu.make_async_copy(
                bufs[i].at[pl.ds(0, szs[k])],
                out_hbm.at[pl.ds(base + offs[k], szs[k])],
                osems[i])
            h.start()
            return h

        g = {0: start_gather(0)}
        o = {}
        for k in range(n):
            if k + 1 < n:
                if k - 1 in o:
                    o[k - 1].wait()
                g[k + 1] = start_gather(k + 1)
            g[k].wait()
            o[k] = start_out(k)
        o[n - 1].wait()
        if n >= 2:
            o[n - 2].wait()

    return gather_kernel


def _ln_body(gath_ref, seg_ref, pos_ref, segtab_ref, gamma_ref, beta_ref,
             out_ref):
    x = gath_ref[...]
    sid = seg_ref[...].astype(jnp.float32)
    pos = pos_ref[...]
    st = segtab_ref[...]
    seg = st[0][None, :] + sid * (st[1] - st[0])[None, :]
    comb = x + pos + seg
    mean = jnp.mean(comb, axis=-1, keepdims=True)
    var = jnp.mean(jnp.square(comb - mean), axis=-1, keepdims=True)
    xhat = (comb - mean) * lax.rsqrt(var + EPS)
    out_ref[...] = xhat * gamma_ref[...][None, :] + beta_ref[...][None, :]


def _ln_call(gath2, seg_ids2, pos_blk, seg_table, gamma, beta, r_blk):
    n, d = gath2.shape
    grid = (n // r_blk,)
    return pl.pallas_call(
        _ln_body,
        grid=grid,
        in_specs=[
            pl.BlockSpec((r_blk, d), lambda i: (i, 0)),
            pl.BlockSpec((r_blk, 1), lambda i: (i, 0)),
            pl.BlockSpec((r_blk, d), lambda i: (0, 0)),
            pl.BlockSpec((2, d), lambda i: (0, 0)),
            pl.BlockSpec((d,), lambda i: (0,)),
            pl.BlockSpec((d,), lambda i: (0,)),
        ],
        out_specs=pl.BlockSpec((r_blk, d), lambda i: (i, 0)),
        out_shape=jax.ShapeDtypeStruct((n, d), jnp.float32),
    )(gath2, seg_ids2, pos_blk, seg_table, gamma, beta)


def kernel(tokens, segment_ids, token_table, pos_table, seg_table, gamma,
           beta):
    b, l = tokens.shape
    d = token_table.shape[1]
    r_blk = 16 * l
    flat = tokens.reshape(-1).astype(jnp.int32)
    gathered = _make_sc_gather(b * l, d)(token_table, flat)
    pos_blk = jnp.tile(pos_table[:l], (r_blk // l, 1))
    out2 = _ln_call(gathered, segment_ids.reshape(-1, 1).astype(jnp.int32),
                    pos_blk, seg_table, gamma, beta, r_blk)
    return out2.reshape(b, l, d)


# speedup vs baseline: 1.0725x; 1.0725x over previous
import functools

import jax
import jax.numpy as jnp
from jax import lax
from jax.experimental import pallas as pl
from jax.experimental.pallas import tpu as pltpu
from jax.experimental.pallas import tpu_sc as plsc

D_MODEL = 128
EPS = 1e-5
_CHUNK = 128


def _make_sc_gather(n_rows: int, d: int):
    info = plsc.get_sparse_core_info()
    nc, ns = info.num_cores, info.num_subcores
    nw = nc * ns
    assert n_rows % nw == 0
    b_per_w = n_rows // nw
    offs = list(range(0, b_per_w, _CHUNK))
    szs = [min(_CHUNK, b_per_w - o) for o in offs]
    n = len(offs)
    mesh = plsc.VectorSubcoreMesh(core_axis_name="c", subcore_axis_name="s")

    @functools.partial(
        pl.kernel,
        mesh=mesh,
        out_type=jax.ShapeDtypeStruct((n_rows, d), jnp.float32),
        scratch_types=[
            pltpu.VMEM((b_per_w,), jnp.int32),
            pltpu.VMEM((_CHUNK, d), jnp.float32),
            pltpu.VMEM((_CHUNK, d), jnp.float32),
            pltpu.SemaphoreType.DMA,
            pltpu.SemaphoreType.DMA,
            pltpu.SemaphoreType.DMA,
            pltpu.SemaphoreType.DMA,
        ],
    )
    def gather_kernel(table_hbm, idx_hbm, out_hbm, idx_v, buf_a, buf_b,
                      gs_a, gs_b, os_a, os_b):
        wid = lax.axis_index("s") * nc + lax.axis_index("c")
        base = wid * b_per_w
        pltpu.sync_copy(idx_hbm.at[pl.ds(base, b_per_w)], idx_v)
        bufs = (buf_a, buf_b)
        gsems = (gs_a, gs_b)
        osems = (os_a, os_b)

        def start_gather(k):
            i = k % 2
            h = pltpu.make_async_copy(
                table_hbm.at[idx_v.at[pl.ds(offs[k], szs[k])]],
                bufs[i].at[pl.ds(0, szs[k])],
                gsems[i])
            h.start()
            return h

        def start_out(k):
            i = k % 2
            h = pltpu.make_async_copy(
                bufs[i].at[pl.ds(0, szs[k])],
                out_hbm.at[pl.ds(base + offs[k], szs[k])],
                osems[i])
            h.start()
            return h

        g = {0: start_gather(0)}
        o = {}
        for k in range(n):
            if k + 1 < n:
                if k - 1 in o:
                    o[k - 1].wait()
                g[k + 1] = start_gather(k + 1)
            g[k].wait()
            o[k] = start_out(k)
        o[n - 1].wait()
        if n >= 2:
            o[n - 2].wait()

    return gather_kernel


def _ln_body(gath_ref, seg_ref, pos_ref, segtab_ref, gamma_ref, beta_ref,
             out_ref):
    x = gath_ref[...]
    sid = seg_ref[...].astype(jnp.float32)
    pos = pos_ref[...]
    st = segtab_ref[...]
    seg = st[0][None, None, :] + sid[:, :, None] * (st[1] - st[0])[None, None, :]
    comb = x + pos[None, :, :] + seg
    mean = jnp.mean(comb, axis=-1, keepdims=True)
    var = jnp.mean(jnp.square(comb - mean), axis=-1, keepdims=True)
    xhat = (comb - mean) * lax.rsqrt(var + EPS)
    out_ref[...] = (xhat * gamma_ref[...][None, None, :]
                    + beta_ref[...][None, None, :])


def _ln_call(gath3, seg_ids, pos, seg_table, gamma, beta, b_blk=64):
    b, l, d = gath3.shape
    grid = (b // b_blk,)
    return pl.pallas_call(
        _ln_body,
        grid=grid,
        in_specs=[
            pl.BlockSpec((b_blk, l, d), lambda i: (i, 0, 0)),
            pl.BlockSpec((b_blk, l), lambda i: (i, 0)),
            pl.BlockSpec((l, d), lambda i: (0, 0)),
            pl.BlockSpec((2, d), lambda i: (0, 0)),
            pl.BlockSpec((d,), lambda i: (0,)),
            pl.BlockSpec((d,), lambda i: (0,)),
        ],
        out_specs=pl.BlockSpec((b_blk, l, d), lambda i: (i, 0, 0)),
        out_shape=jax.ShapeDtypeStruct((b, l, d), jnp.float32),
    )(gath3, seg_ids, pos, seg_table, gamma, beta)


_N_SLICES = 4


def kernel(tokens, segment_ids, token_table, pos_table, seg_table, gamma,
           beta):
    b, l = tokens.shape
    d = token_table.shape[1]
    seg32 = segment_ids.astype(jnp.int32)
    bs = b // _N_SLICES
    gather_fn = _make_sc_gather(bs * l, d)
    outs = []
    for s in range(_N_SLICES):
        tok_s = tokens[s * bs:(s + 1) * bs].reshape(-1).astype(jnp.int32)
        gath = gather_fn(token_table, tok_s).reshape(bs, l, d)
        outs.append(_ln_call(gath, seg32[s * bs:(s + 1) * bs],
                             pos_table[:l], seg_table, gamma, beta))
    return jnp.concatenate(outs, axis=0)
